# baseline (speedup 1.0000x reference)
import functools

import jax
import jax.numpy as jnp
from jax import lax
from jax.experimental import pallas as pl
from jax.experimental.pallas import tpu as pltpu
from jax.experimental.pallas import tpu_sc as plsc

_NC = 2
_NS = 16
_C = 128


def _mesh():
    return plsc.VectorSubcoreMesh(core_axis_name="c", subcore_axis_name="s")


def _sc_gather(table, idx2d):
    R, C = idx2d.shape
    D = table.shape[1]
    NW = _NC * _NS
    nfull, nrem = R // NW, R % NW

    @functools.partial(
        pl.kernel,
        out_type=jax.ShapeDtypeStruct((R, C, D), jnp.float32),
        mesh=_mesh(),
        scratch_types=[
            pltpu.VMEM((2, C), jnp.int32),
            pltpu.VMEM((2, C, D), jnp.float32),
            pltpu.SemaphoreType.DMA,
            pltpu.SemaphoreType.DMA,
            pltpu.SemaphoreType.DMA,
        ],
    )
    def gk(table_hbm, idx_hbm, out_hbm, idxb, rows, semi, semg, semw):
        wid = lax.axis_index("s") * _NC + lax.axis_index("c")
        start = wid * nfull + jnp.minimum(wid, nrem)
        nj = nfull + (wid < nrem).astype(jnp.int32)
        pltpu.sync_copy(idx_hbm.at[start], idxb.at[0])
        pltpu.async_copy(table_hbm.at[idxb.at[0]], rows.at[0], semg)

        def step(j, carry):
            par = j % 2

            @pl.when(j + 1 < nj)
            def _():
                pltpu.async_copy(idx_hbm.at[start + j + 1], idxb.at[1 - par],
                                 semi)

            pltpu.make_async_copy(table_hbm.at[idxb.at[0]], rows.at[0],
                                  semg).wait()
            pltpu.async_copy(rows.at[par], out_hbm.at[start + j], semw)

            @pl.when(j + 1 < nj)
            def _():
                pltpu.make_async_copy(idx_hbm.at[start], idxb.at[0],
                                      semi).wait()

                @pl.when(j >= 1)
                def _():
                    pltpu.make_async_copy(rows.at[0], out_hbm.at[start],
                                          semw).wait()

                pltpu.async_copy(table_hbm.at[idxb.at[1 - par]],
                                 rows.at[1 - par], semg)

            return carry

        lax.fori_loop(0, nj, step, 0)
        pltpu.make_async_copy(rows.at[0], out_hbm.at[start], semw).wait()

        @pl.when(nj >= 2)
        def _():
            pltpu.make_async_copy(rows.at[0], out_hbm.at[start], semw).wait()

    return gk(table, idx2d)


def _sc_scatter_add(e2d, rid_flat, Nm):
    E, D = e2d.shape
    NW = _NC * _NS
    own = (Nm + NW - 1) // NW
    rpt = ((own + 1 + 7) // 8) * 8
    Ep = rid_flat.shape[0]
    OUTER = Ep // 2048
    zeros = jnp.zeros((rpt, D), jnp.float32)

    @functools.partial(
        pl.kernel,
        out_type=jax.ShapeDtypeStruct((NW, rpt, D), jnp.float32),
        mesh=_mesh(),
        scratch_types=[
            pltpu.VMEM((2048,), jnp.int32),
            pltpu.VMEM((rpt, D), jnp.float32),
            pltpu.VMEM((160,), jnp.int32),
            pltpu.VMEM((16,), jnp.int32),
            pltpu.VMEM((2, 16, D), jnp.float32),
            pltpu.SemaphoreType.DMA,
        ],
        compiler_params=pltpu.CompilerParams(needs_layout_passes=False),
    )
    def sk(zeros_hbm, e_hbm, idx_hbm, out_hbm, idxbuf, acc, plist, midx,
           grow, sem):
        c = lax.axis_index("c")
        s = lax.axis_index("s")
        wid = s * _NC + c
        base = wid * own
        iota16 = lax.iota(jnp.int32, 16)
        pltpu.sync_copy(zeros_hbm, acc)

        def drain(m):
            n16 = m // 16

            @pl.when(n16 > 0)
            def _():
                midx[...] = plist[pl.ds(0, 16)] >> 9
                pltpu.async_copy(e_hbm.at[midx], grow.at[0], sem)

                def batch(b, carry):
                    par = b % 2
                    pltpu.make_async_copy(e_hbm.at[midx], grow.at[0],
                                          sem).wait()

                    @pl.when(b + 1 < n16)
                    def _():
                        midx[...] = plist[pl.ds(b * 16 + 16, 16)] >> 9
                        pltpu.async_copy(e_hbm.at[midx], grow.at[1 - par],
                                         sem)

                    pk = plist[pl.ds(b * 16, 16)]
                    for mm in range(16):
                        li = pk[mm] & 511
                        for k in range(D // 16):
                            plsc.addupdate(acc.at[li, pl.ds(k * 16, 16)],
                                           grow[par, mm, pl.ds(k * 16, 16)])
                    return carry

                lax.fori_loop(0, n16, batch, 0)
                plist[pl.ds(0, 16)] = plist[pl.ds(n16 * 16, 16)]

            return m - n16 * 16

        def outer(o, m):
            pltpu.sync_copy(idx_hbm.at[pl.ds(o * 2048, 2048)], idxbuf)

            def rnd(r, m):
                for vv in range(8):
                    iv = idxbuf[pl.ds(r * 128 + vv * 16, 16)]
                    li = iv - base
                    ms = (li >= 0) & (li < own)

                    def compact(mm_):
                        pos = (o * 2048 + r * 128 + vv * 16) + iota16
                        packed = (pos << 9) | jnp.where(ms, li, 0)
                        cs = plsc.cumsum(ms.astype(jnp.int32))
                        plsc.store_scatter(plist, [mm_ + cs - 1], packed,
                                           mask=ms)
                        return mm_ + cs[15]

                    m = lax.cond(jnp.any(ms), compact, lambda mm_: mm_, m)
                return drain(m)

            return lax.fori_loop(0, 16, rnd, m)

        m = lax.fori_loop(0, OUTER, outer, jnp.int32(0))
        plist[pl.ds(m, 16)] = jnp.full((16,), own, jnp.int32)
        drain(((m + 15) // 16) * 16)
        pltpu.sync_copy(acc, out_hbm.at[wid])

    out = sk(zeros, e2d, rid_flat)
    return out[:, :own].reshape(NW * own, D)[:Nm]


def _node_proj(v2, Ws, Wr):
    Nm, H = v2.shape
    NB = 1000
    grid = (Nm // NB,)

    def body(v_ref, ws_ref, wr_ref, os_ref, or_ref):
        v = v_ref[...]
        os_ref[...] = jnp.dot(v, ws_ref[...], preferred_element_type=jnp.float32)
        or_ref[...] = jnp.dot(v, wr_ref[...], preferred_element_type=jnp.float32)

    return pl.pallas_call(
        body,
        grid=grid,
        in_specs=[
            pl.BlockSpec((NB, H), lambda i: (i, 0)),
            pl.BlockSpec((H, H), lambda i: (0, 0)),
            pl.BlockSpec((H, H), lambda i: (0, 0)),
        ],
        out_specs=[
            pl.BlockSpec((NB, H), lambda i: (i, 0)),
            pl.BlockSpec((NB, H), lambda i: (i, 0)),
        ],
        out_shape=[
            jax.ShapeDtypeStruct((Nm, H), jnp.float32),
            jax.ShapeDtypeStruct((Nm, H), jnp.float32),
        ],
    )(v2, Ws, Wr)


def _mlp_ln_body(x, extra, w1_ref, w2_ref, b1_ref, g1_ref, be1_ref, b2_ref):
    pre = jnp.dot(x, w1_ref[...], preferred_element_type=jnp.float32)
    pre = pre + extra + b1_ref[...]
    h = pre * jax.nn.sigmoid(pre)
    mu = jnp.mean(h, axis=-1, keepdims=True)
    var = jnp.mean((h - mu) ** 2, axis=-1, keepdims=True)
    h = (h - mu) * lax.rsqrt(var + 1e-5) * g1_ref[...] + be1_ref[...]
    return x + jnp.dot(h, w2_ref[...], preferred_element_type=jnp.float32) + b2_ref[...]


def _edge_mlp(e2, gs, gr, W1e, W2, b1, g1, be1, b2):
    E, H = e2.shape
    EB = 640
    grid = (E // EB,)

    def body(e_ref, gs_ref, gr_ref, w1_ref, w2_ref, b1_ref, g1_ref, be1_ref,
             b2_ref, o_ref):
        o_ref[...] = _mlp_ln_body(e_ref[...], gs_ref[...] + gr_ref[...],
                                  w1_ref, w2_ref, b1_ref, g1_ref, be1_ref, b2_ref)

    row = pl.BlockSpec((EB, H), lambda i: (i, 0))
    mat = pl.BlockSpec((H, H), lambda i: (0, 0))
    vec = pl.BlockSpec((1, H), lambda i: (0, 0))
    return pl.pallas_call(
        body,
        grid=grid,
        in_specs=[row, row, row, mat, mat, vec, vec, vec, vec],
        out_specs=row,
        out_shape=jax.ShapeDtypeStruct((E, H), jnp.float32),
    )(e2, gs, gr, W1e, W2, b1, g1, be1, b2)


def _node_mlp(v2, agg, W1v, W1a, W2, b1, g1, be1, b2):
    Nm, H = v2.shape
    NB = 1000
    grid = (Nm // NB,)

    def body(v_ref, a_ref, w1_ref, w1a_ref, w2_ref, b1_ref, g1_ref,
             be1_ref, b2_ref, o_ref):
        extra = jnp.dot(a_ref[...], w1a_ref[...],
                        preferred_element_type=jnp.float32)
        o_ref[...] = _mlp_ln_body(v_ref[...], extra, w1_ref, w2_ref, b1_ref,
                                  g1_ref, be1_ref, b2_ref)

    row = pl.BlockSpec((NB, H), lambda i: (i, 0))
    mat = pl.BlockSpec((H, H), lambda i: (0, 0))
    vec = pl.BlockSpec((1, H), lambda i: (0, 0))
    return pl.pallas_call(
        body,
        grid=grid,
        in_specs=[row, row, mat, mat, mat, vec, vec, vec, vec],
        out_specs=row,
        out_shape=jax.ShapeDtypeStruct((Nm, H), jnp.float32),
    )(v2, agg, W1v, W1a, W2, b1, g1, be1, b2)


def kernel(vM, eM, senders, receivers, W1m, b1m, g1m, be1m, W2m, b2m,
           W1n, b1n, g1n, be1n, W2n, b2n):
    B, Nm, H = vM.shape
    E = eM.shape[1]
    v2 = vM[0]
    e2 = eM[0]
    sid = senders.astype(jnp.int32).reshape(E // _C, _C)
    rid = receivers.astype(jnp.int32).reshape(E // _C, _C)

    b1m_ = b1m.reshape(1, H)
    g1m_ = g1m.reshape(1, H)
    be1m_ = be1m.reshape(1, H)
    b2m_ = b2m.reshape(1, H)
    b1n_ = b1n.reshape(1, H)
    g1n_ = g1n.reshape(1, H)
    be1n_ = be1n.reshape(1, H)
    b2n_ = b2n.reshape(1, H)

    Ps, Pr = _node_proj(v2, W1m[H:2 * H], W1m[2 * H:])
    gs = _sc_gather(Ps, sid).reshape(E, H)
    gr = _sc_gather(Pr, rid).reshape(E, H)
    e2out = _edge_mlp(e2, gs, gr, W1m[:H], W2m, b1m_, g1m_, be1m_, b2m_)
    r32 = receivers.astype(jnp.int32)
    Ep = ((E + 2047) // 2048) * 2048
    rid_flat = jnp.concatenate(
        [r32, jnp.full((Ep - E,), 2 ** 20, jnp.int32)])
    agg = _sc_scatter_add(e2out, rid_flat, Nm)
    v2out = _node_mlp(v2, agg, W1n[:H], W1n[H:], W2n,
                      b1n_, g1n_, be1n_, b2n_)
    return (v2out.reshape(B, Nm, H), e2out.reshape(B, E, H))


# speedup vs baseline: 1.1100x; 1.1100x over previous
import functools

import jax
import jax.numpy as jnp
from jax import lax
from jax.experimental import pallas as pl
from jax.experimental.pallas import tpu as pltpu
from jax.experimental.pallas import tpu_sc as plsc

_NC = 2
_NS = 16
_C = 128


def _mesh():
    return plsc.VectorSubcoreMesh(core_axis_name="c", subcore_axis_name="s")


def _sc_gather(table, idx2d):
    R, C = idx2d.shape
    D = table.shape[1]
    NW = _NC * _NS
    nfull, nrem = R // NW, R % NW

    @functools.partial(
        pl.kernel,
        out_type=jax.ShapeDtypeStruct((R, C, D), jnp.float32),
        mesh=_mesh(),
        scratch_types=[
            pltpu.VMEM((2, C), jnp.int32),
            pltpu.VMEM((2, C, D), jnp.float32),
            pltpu.SemaphoreType.DMA,
            pltpu.SemaphoreType.DMA,
            pltpu.SemaphoreType.DMA,
        ],
    )
    def gk(table_hbm, idx_hbm, out_hbm, idxb, rows, semi, semg, semw):
        wid = lax.axis_index("s") * _NC + lax.axis_index("c")
        start = wid * nfull + jnp.minimum(wid, nrem)
        nj = nfull + (wid < nrem).astype(jnp.int32)
        pltpu.sync_copy(idx_hbm.at[start], idxb.at[0])
        pltpu.async_copy(table_hbm.at[idxb.at[0]], rows.at[0], semg)

        def step(j, carry):
            par = j % 2

            @pl.when(j + 1 < nj)
            def _():
                pltpu.async_copy(idx_hbm.at[start + j + 1], idxb.at[1 - par],
                                 semi)

            pltpu.make_async_copy(table_hbm.at[idxb.at[0]], rows.at[0],
                                  semg).wait()
            pltpu.async_copy(rows.at[par], out_hbm.at[start + j], semw)

            @pl.when(j + 1 < nj)
            def _():
                pltpu.make_async_copy(idx_hbm.at[start], idxb.at[0],
                                      semi).wait()

                @pl.when(j >= 1)
                def _():
                    pltpu.make_async_copy(rows.at[0], out_hbm.at[start],
                                          semw).wait()

                pltpu.async_copy(table_hbm.at[idxb.at[1 - par]],
                                 rows.at[1 - par], semg)

            return carry

        lax.fori_loop(0, nj, step, 0)
        pltpu.make_async_copy(rows.at[0], out_hbm.at[start], semw).wait()

        @pl.when(nj >= 2)
        def _():
            pltpu.make_async_copy(rows.at[0], out_hbm.at[start], semw).wait()

    return gk(table, idx2d)


def _sc_scatter_add(e2d, rid_flat, Nm):
    E, D = e2d.shape
    NW = _NC * _NS
    own = (Nm + NW - 1) // NW
    rpt = ((own + 1 + 7) // 8) * 8
    Ep = rid_flat.shape[0]
    OUTER = Ep // 2048
    zeros = jnp.zeros((rpt, D), jnp.float32)

    @functools.partial(
        pl.kernel,
        out_type=jax.ShapeDtypeStruct((NW, rpt, D), jnp.float32),
        mesh=_mesh(),
        scratch_types=[
            pltpu.VMEM((2048,), jnp.int32),
            pltpu.VMEM((rpt, D), jnp.float32),
            pltpu.VMEM((160,), jnp.int32),
            pltpu.VMEM((16,), jnp.int32),
            pltpu.VMEM((2, 16, D), jnp.float32),
            pltpu.SemaphoreType.DMA,
        ],
        compiler_params=pltpu.CompilerParams(needs_layout_passes=False),
    )
    def sk(zeros_hbm, e_hbm, idx_hbm, out_hbm, idxbuf, acc, plist, midx,
           grow, sem):
        c = lax.axis_index("c")
        s = lax.axis_index("s")
        wid = s * _NC + c
        base = wid * own
        iota16 = lax.iota(jnp.int32, 16)
        pltpu.sync_copy(zeros_hbm, acc)

        def drain(m):
            n16 = m // 16

            @pl.when(n16 > 0)
            def _():
                midx[...] = plist[pl.ds(0, 16)] >> 9
                pltpu.async_copy(e_hbm.at[midx], grow.at[0], sem)

                def batch(b, carry):
                    par = b % 2
                    pltpu.make_async_copy(e_hbm.at[midx], grow.at[0],
                                          sem).wait()

                    @pl.when(b + 1 < n16)
                    def _():
                        midx[...] = plist[pl.ds(b * 16 + 16, 16)] >> 9
                        pltpu.async_copy(e_hbm.at[midx], grow.at[1 - par],
                                         sem)

                    pk = plist[pl.ds(b * 16, 16)]
                    for mm in range(16):
                        li = pk[mm] & 511
                        for k in range(D // 16):
                            plsc.addupdate(acc.at[li, pl.ds(k * 16, 16)],
                                           grow[par, mm, pl.ds(k * 16, 16)])
                    return carry

                lax.fori_loop(0, n16, batch, 0)
                plist[pl.ds(0, 16)] = plist[pl.ds(n16 * 16, 16)]

            return m - n16 * 16

        def outer(o, m):
            pltpu.sync_copy(idx_hbm.at[pl.ds(o * 2048, 2048)], idxbuf)

            def rnd(r, m):
                for vv in range(8):
                    iv = idxbuf[pl.ds(r * 128 + vv * 16, 16)]
                    li = iv - base
                    ms = (li >= 0) & (li < own)

                    pos = (o * 2048 + r * 128 + vv * 16) + iota16
                    packed = (pos << 9) | jnp.where(ms, li, 0)
                    cs = plsc.cumsum(ms.astype(jnp.int32))
                    plsc.store_scatter(plist, [m + cs - 1], packed, mask=ms)
                    m = m + cs[15]
                return drain(m)

            return lax.fori_loop(0, 16, rnd, m)

        m = lax.fori_loop(0, OUTER, outer, jnp.int32(0))
        plist[pl.ds(m, 16)] = jnp.full((16,), own, jnp.int32)
        drain(((m + 15) // 16) * 16)
        pltpu.sync_copy(acc, out_hbm.at[wid])

    out = sk(zeros, e2d, rid_flat)
    return out[:, :own].reshape(NW * own, D)[:Nm]


def _node_proj(v2, Ws, Wr):
    Nm, H = v2.shape
    NB = 1000
    grid = (Nm // NB,)

    def body(v_ref, ws_ref, wr_ref, os_ref, or_ref):
        v = v_ref[...]
        os_ref[...] = jnp.dot(v, ws_ref[...], preferred_element_type=jnp.float32)
        or_ref[...] = jnp.dot(v, wr_ref[...], preferred_element_type=jnp.float32)

    return pl.pallas_call(
        body,
        grid=grid,
        in_specs=[
            pl.BlockSpec((NB, H), lambda i: (i, 0)),
            pl.BlockSpec((H, H), lambda i: (0, 0)),
            pl.BlockSpec((H, H), lambda i: (0, 0)),
        ],
        out_specs=[
            pl.BlockSpec((NB, H), lambda i: (i, 0)),
            pl.BlockSpec((NB, H), lambda i: (i, 0)),
        ],
        out_shape=[
            jax.ShapeDtypeStruct((Nm, H), jnp.float32),
            jax.ShapeDtypeStruct((Nm, H), jnp.float32),
        ],
    )(v2, Ws, Wr)


def _mlp_ln_body(x, extra, w1_ref, w2_ref, b1_ref, g1_ref, be1_ref, b2_ref):
    pre = jnp.dot(x, w1_ref[...], preferred_element_type=jnp.float32)
    pre = pre + extra + b1_ref[...]
    h = pre * jax.nn.sigmoid(pre)
    mu = jnp.mean(h, axis=-1, keepdims=True)
    var = jnp.mean((h - mu) ** 2, axis=-1, keepdims=True)
    h = (h - mu) * lax.rsqrt(var + 1e-5) * g1_ref[...] + be1_ref[...]
    return x + jnp.dot(h, w2_ref[...], preferred_element_type=jnp.float32) + b2_ref[...]


def _edge_mlp(e2, gs, gr, W1e, W2, b1, g1, be1, b2):
    E, H = e2.shape
    EB = 640
    grid = (E // EB,)

    def body(e_ref, gs_ref, gr_ref, w1_ref, w2_ref, b1_ref, g1_ref, be1_ref,
             b2_ref, o_ref):
        o_ref[...] = _mlp_ln_body(e_ref[...], gs_ref[...] + gr_ref[...],
                                  w1_ref, w2_ref, b1_ref, g1_ref, be1_ref, b2_ref)

    row = pl.BlockSpec((EB, H), lambda i: (i, 0))
    mat = pl.BlockSpec((H, H), lambda i: (0, 0))
    vec = pl.BlockSpec((1, H), lambda i: (0, 0))
    return pl.pallas_call(
        body,
        grid=grid,
        in_specs=[row, row, row, mat, mat, vec, vec, vec, vec],
        out_specs=row,
        out_shape=jax.ShapeDtypeStruct((E, H), jnp.float32),
    )(e2, gs, gr, W1e, W2, b1, g1, be1, b2)


def _node_mlp(v2, agg, W1v, W1a, W2, b1, g1, be1, b2):
    Nm, H = v2.shape
    NB = 1000
    grid = (Nm // NB,)

    def body(v_ref, a_ref, w1_ref, w1a_ref, w2_ref, b1_ref, g1_ref,
             be1_ref, b2_ref, o_ref):
        extra = jnp.dot(a_ref[...], w1a_ref[...],
                        preferred_element_type=jnp.float32)
        o_ref[...] = _mlp_ln_body(v_ref[...], extra, w1_ref, w2_ref, b1_ref,
                                  g1_ref, be1_ref, b2_ref)

    row = pl.BlockSpec((NB, H), lambda i: (i, 0))
    mat = pl.BlockSpec((H, H), lambda i: (0, 0))
    vec = pl.BlockSpec((1, H), lambda i: (0, 0))
    return pl.pallas_call(
        body,
        grid=grid,
        in_specs=[row, row, mat, mat, mat, vec, vec, vec, vec],
        out_specs=row,
        out_shape=jax.ShapeDtypeStruct((Nm, H), jnp.float32),
    )(v2, agg, W1v, W1a, W2, b1, g1, be1, b2)


def kernel(vM, eM, senders, receivers, W1m, b1m, g1m, be1m, W2m, b2m,
           W1n, b1n, g1n, be1n, W2n, b2n):
    B, Nm, H = vM.shape
    E = eM.shape[1]
    v2 = vM[0]
    e2 = eM[0]
    sid = senders.astype(jnp.int32).reshape(E // _C, _C)
    rid = receivers.astype(jnp.int32).reshape(E // _C, _C)

    b1m_ = b1m.reshape(1, H)
    g1m_ = g1m.reshape(1, H)
    be1m_ = be1m.reshape(1, H)
    b2m_ = b2m.reshape(1, H)
    b1n_ = b1n.reshape(1, H)
    g1n_ = g1n.reshape(1, H)
    be1n_ = be1n.reshape(1, H)
    b2n_ = b2n.reshape(1, H)

    Ps, Pr = _node_proj(v2, W1m[H:2 * H], W1m[2 * H:])
    gs = _sc_gather(Ps, sid).reshape(E, H)
    gr = _sc_gather(Pr, rid).reshape(E, H)
    e2out = _edge_mlp(e2, gs, gr, W1m[:H], W2m, b1m_, g1m_, be1m_, b2m_)
    r32 = receivers.astype(jnp.int32)
    Ep = ((E + 2047) // 2048) * 2048
    rid_flat = jnp.concatenate(
        [r32, jnp.full((Ep - E,), 2 ** 20, jnp.int32)])
    agg = _sc_scatter_add(e2out, rid_flat, Nm)
    v2out = _node_mlp(v2, agg, W1n[:H], W1n[H:], W2n,
                      b1n_, g1n_, be1n_, b2n_)
    return (v2out.reshape(B, Nm, H), e2out.reshape(B, E, H))


# speedup vs baseline: 1.1525x; 1.0383x over previous
import functools

import jax
import jax.numpy as jnp
from jax import lax
from jax.experimental import pallas as pl
from jax.experimental.pallas import tpu as pltpu
from jax.experimental.pallas import tpu_sc as plsc

_NC = 2
_NS = 16
_C = 128


def _mesh():
    return plsc.VectorSubcoreMesh(core_axis_name="c", subcore_axis_name="s")


def _sc_gather(table, idx2d):
    R, C = idx2d.shape
    D = table.shape[1]
    NW = _NC * _NS
    nfull, nrem = R // NW, R % NW

    @functools.partial(
        pl.kernel,
        out_type=jax.ShapeDtypeStruct((R, C, D), jnp.float32),
        mesh=_mesh(),
        scratch_types=[
            pltpu.VMEM((2, C), jnp.int32),
            pltpu.VMEM((2, C, D), jnp.float32),
            pltpu.SemaphoreType.DMA,
            pltpu.SemaphoreType.DMA,
            pltpu.SemaphoreType.DMA,
        ],
    )
    def gk(table_hbm, idx_hbm, out_hbm, idxb, rows, semi, semg, semw):
        wid = lax.axis_index("s") * _NC + lax.axis_index("c")
        start = wid * nfull + jnp.minimum(wid, nrem)
        nj = nfull + (wid < nrem).astype(jnp.int32)
        pltpu.sync_copy(idx_hbm.at[start], idxb.at[0])
        pltpu.async_copy(table_hbm.at[idxb.at[0]], rows.at[0], semg)

        def step(j, carry):
            par = j % 2

            @pl.when(j + 1 < nj)
            def _():
                pltpu.async_copy(idx_hbm.at[start + j + 1], idxb.at[1 - par],
                                 semi)

            pltpu.make_async_copy(table_hbm.at[idxb.at[0]], rows.at[0],
                                  semg).wait()
            pltpu.async_copy(rows.at[par], out_hbm.at[start + j], semw)

            @pl.when(j + 1 < nj)
            def _():
                pltpu.make_async_copy(idx_hbm.at[start], idxb.at[0],
                                      semi).wait()

                @pl.when(j >= 1)
                def _():
                    pltpu.make_async_copy(rows.at[0], out_hbm.at[start],
                                          semw).wait()

                pltpu.async_copy(table_hbm.at[idxb.at[1 - par]],
                                 rows.at[1 - par], semg)

            return carry

        lax.fori_loop(0, nj, step, 0)
        pltpu.make_async_copy(rows.at[0], out_hbm.at[start], semw).wait()

        @pl.when(nj >= 2)
        def _():
            pltpu.make_async_copy(rows.at[0], out_hbm.at[start], semw).wait()

    return gk(table, idx2d)


def _sc_scatter_add(e2d, rid_flat, Nm):
    E, D = e2d.shape
    NW = _NC * _NS
    own = (Nm + NW - 1) // NW
    rpt = ((own + 1 + 7) // 8) * 8
    Ep = rid_flat.shape[0]
    OUTER = Ep // 2048
    zeros = jnp.zeros((rpt, D), jnp.float32)

    @functools.partial(
        pl.kernel,
        out_type=jax.ShapeDtypeStruct((NW, rpt, D), jnp.float32),
        mesh=_mesh(),
        scratch_types=[
            pltpu.VMEM((2, 2048), jnp.int32),
            pltpu.VMEM((rpt, D), jnp.float32),
            pltpu.VMEM((160,), jnp.int32),
            pltpu.VMEM((16,), jnp.int32),
            pltpu.VMEM((2, 16, D), jnp.float32),
            pltpu.SemaphoreType.DMA,
            pltpu.SemaphoreType.DMA,
        ],
        compiler_params=pltpu.CompilerParams(needs_layout_passes=False),
    )
    def sk(zeros_hbm, e_hbm, idx_hbm, out_hbm, idxbuf, acc, plist, midx,
           grow, sem, semx):
        c = lax.axis_index("c")
        s = lax.axis_index("s")
        wid = s * _NC + c
        base = wid * own
        iota16 = lax.iota(jnp.int32, 16)
        pltpu.sync_copy(zeros_hbm, acc)

        def drain(m):
            n16 = m // 16

            @pl.when(n16 > 0)
            def _():
                midx[...] = plist[pl.ds(0, 16)] >> 9
                pltpu.async_copy(e_hbm.at[midx], grow.at[0], sem)

                def batch(b, carry):
                    par = b % 2
                    pltpu.make_async_copy(e_hbm.at[midx], grow.at[0],
                                          sem).wait()

                    @pl.when(b + 1 < n16)
                    def _():
                        midx[...] = plist[pl.ds(b * 16 + 16, 16)] >> 9
                        pltpu.async_copy(e_hbm.at[midx], grow.at[1 - par],
                                         sem)

                    pk = plist[pl.ds(b * 16, 16)]
                    for mm in range(16):
                        li = pk[mm] & 511
                        for k in range(D // 16):
                            plsc.addupdate(acc.at[li, pl.ds(k * 16, 16)],
                                           grow[par, mm, pl.ds(k * 16, 16)])
                    return carry

                lax.fori_loop(0, n16, batch, 0)
                plist[pl.ds(0, 16)] = plist[pl.ds(n16 * 16, 16)]

            return m - n16 * 16

        pltpu.async_copy(idx_hbm.at[pl.ds(0, 2048)], idxbuf.at[0], semx)

        def outer(o, m):
            op = o % 2
            pltpu.make_async_copy(idx_hbm.at[pl.ds(0, 2048)], idxbuf.at[0],
                                  semx).wait()

            @pl.when(o + 1 < OUTER)
            def _():
                pltpu.async_copy(idx_hbm.at[pl.ds((o + 1) * 2048, 2048)],
                                 idxbuf.at[1 - op], semx)

            def rnd(r, m):
                for vv in range(8):
                    iv = idxbuf[op, pl.ds(r * 128 + vv * 16, 16)]
                    li = iv - base
                    ms = (li >= 0) & (li < own)

                    pos = (o * 2048 + r * 128 + vv * 16) + iota16
                    packed = (pos << 9) | jnp.where(ms, li, 0)
                    cs = plsc.cumsum(ms.astype(jnp.int32))
                    plsc.store_scatter(plist, [m + cs - 1], packed, mask=ms)
                    m = m + plsc.all_reduce_population_count(ms)[0]
                return drain(m)

            return lax.fori_loop(0, 16, rnd, m)

        m = lax.fori_loop(0, OUTER, outer, jnp.int32(0))
        plist[pl.ds(m, 16)] = jnp.full((16,), own, jnp.int32)
        drain(((m + 15) // 16) * 16)
        pltpu.sync_copy(acc, out_hbm.at[wid])

    out = sk(zeros, e2d, rid_flat)
    return out[:, :own].reshape(NW * own, D)[:Nm]


def _node_proj(v2, Ws, Wr):
    Nm, H = v2.shape
    NB = 1000
    grid = (Nm // NB,)

    def body(v_ref, ws_ref, wr_ref, os_ref, or_ref):
        v = v_ref[...]
        os_ref[...] = jnp.dot(v, ws_ref[...], preferred_element_type=jnp.float32)
        or_ref[...] = jnp.dot(v, wr_ref[...], preferred_element_type=jnp.float32)

    return pl.pallas_call(
        body,
        grid=grid,
        in_specs=[
            pl.BlockSpec((NB, H), lambda i: (i, 0)),
            pl.BlockSpec((H, H), lambda i: (0, 0)),
            pl.BlockSpec((H, H), lambda i: (0, 0)),
        ],
        out_specs=[
            pl.BlockSpec((NB, H), lambda i: (i, 0)),
            pl.BlockSpec((NB, H), lambda i: (i, 0)),
        ],
        out_shape=[
            jax.ShapeDtypeStruct((Nm, H), jnp.float32),
            jax.ShapeDtypeStruct((Nm, H), jnp.float32),
        ],
    )(v2, Ws, Wr)


def _mlp_ln_body(x, extra, w1_ref, w2_ref, b1_ref, g1_ref, be1_ref, b2_ref):
    pre = jnp.dot(x, w1_ref[...], preferred_element_type=jnp.float32)
    pre = pre + extra + b1_ref[...]
    h = pre * jax.nn.sigmoid(pre)
    mu = jnp.mean(h, axis=-1, keepdims=True)
    var = jnp.mean((h - mu) ** 2, axis=-1, keepdims=True)
    h = (h - mu) * lax.rsqrt(var + 1e-5) * g1_ref[...] + be1_ref[...]
    return x + jnp.dot(h, w2_ref[...], preferred_element_type=jnp.float32) + b2_ref[...]


def _edge_mlp(e2, gs, gr, W1e, W2, b1, g1, be1, b2):
    E, H = e2.shape
    EB = 640
    grid = (E // EB,)

    def body(e_ref, gs_ref, gr_ref, w1_ref, w2_ref, b1_ref, g1_ref, be1_ref,
             b2_ref, o_ref):
        o_ref[...] = _mlp_ln_body(e_ref[...], gs_ref[...] + gr_ref[...],
                                  w1_ref, w2_ref, b1_ref, g1_ref, be1_ref, b2_ref)

    row = pl.BlockSpec((EB, H), lambda i: (i, 0))
    mat = pl.BlockSpec((H, H), lambda i: (0, 0))
    vec = pl.BlockSpec((1, H), lambda i: (0, 0))
    return pl.pallas_call(
        body,
        grid=grid,
        in_specs=[row, row, row, mat, mat, vec, vec, vec, vec],
        out_specs=row,
        out_shape=jax.ShapeDtypeStruct((E, H), jnp.float32),
    )(e2, gs, gr, W1e, W2, b1, g1, be1, b2)


def _node_mlp(v2, agg, W1v, W1a, W2, b1, g1, be1, b2):
    Nm, H = v2.shape
    NB = 1000
    grid = (Nm // NB,)

    def body(v_ref, a_ref, w1_ref, w1a_ref, w2_ref, b1_ref, g1_ref,
             be1_ref, b2_ref, o_ref):
        extra = jnp.dot(a_ref[...], w1a_ref[...],
                        preferred_element_type=jnp.float32)
        o_ref[...] = _mlp_ln_body(v_ref[...], extra, w1_ref, w2_ref, b1_ref,
                                  g1_ref, be1_ref, b2_ref)

    row = pl.BlockSpec((NB, H), lambda i: (i, 0))
    mat = pl.BlockSpec((H, H), lambda i: (0, 0))
    vec = pl.BlockSpec((1, H), lambda i: (0, 0))
    return pl.pallas_call(
        body,
        grid=grid,
        in_specs=[row, row, mat, mat, mat, vec, vec, vec, vec],
        out_specs=row,
        out_shape=jax.ShapeDtypeStruct((Nm, H), jnp.float32),
    )(v2, agg, W1v, W1a, W2, b1, g1, be1, b2)


def kernel(vM, eM, senders, receivers, W1m, b1m, g1m, be1m, W2m, b2m,
           W1n, b1n, g1n, be1n, W2n, b2n):
    B, Nm, H = vM.shape
    E = eM.shape[1]
    v2 = vM[0]
    e2 = eM[0]
    sid = senders.astype(jnp.int32).reshape(E // _C, _C)
    rid = receivers.astype(jnp.int32).reshape(E // _C, _C)

    b1m_ = b1m.reshape(1, H)
    g1m_ = g1m.reshape(1, H)
    be1m_ = be1m.reshape(1, H)
    b2m_ = b2m.reshape(1, H)
    b1n_ = b1n.reshape(1, H)
    g1n_ = g1n.reshape(1, H)
    be1n_ = be1n.reshape(1, H)
    b2n_ = b2n.reshape(1, H)

    Ps, Pr = _node_proj(v2, W1m[H:2 * H], W1m[2 * H:])
    gs = _sc_gather(Ps, sid).reshape(E, H)
    gr = _sc_gather(Pr, rid).reshape(E, H)
    e2out = _edge_mlp(e2, gs, gr, W1m[:H], W2m, b1m_, g1m_, be1m_, b2m_)
    r32 = receivers.astype(jnp.int32)
    Ep = ((E + 2047) // 2048) * 2048
    rid_flat = jnp.concatenate(
        [r32, jnp.full((Ep - E,), 2 ** 20, jnp.int32)])
    agg = _sc_scatter_add(e2out, rid_flat, Nm)
    v2out = _node_mlp(v2, agg, W1n[:H], W1n[H:], W2n,
                      b1n_, g1n_, be1n_, b2n_)
    return (v2out.reshape(B, Nm, H), e2out.reshape(B, E, H))


# speedup vs baseline: 1.3901x; 1.2062x over previous
import functools

import jax
import jax.numpy as jnp
from jax import lax
from jax.experimental import pallas as pl
from jax.experimental.pallas import tpu as pltpu
from jax.experimental.pallas import tpu_sc as plsc

_NC = 2
_NS = 16
_C = 128


def _mesh():
    return plsc.VectorSubcoreMesh(core_axis_name="c", subcore_axis_name="s")


def _sc_gather(table, idx2d):
    R, C = idx2d.shape
    D = table.shape[1]
    NW = _NC * _NS
    nfull, nrem = R // NW, R % NW

    @functools.partial(
        pl.kernel,
        out_type=jax.ShapeDtypeStruct((R, C, D), jnp.float32),
        mesh=_mesh(),
        scratch_types=[
            pltpu.VMEM((2, C), jnp.int32),
            pltpu.VMEM((2, C, D), jnp.float32),
            pltpu.SemaphoreType.DMA,
            pltpu.SemaphoreType.DMA,
            pltpu.SemaphoreType.DMA,
        ],
    )
    def gk(table_hbm, idx_hbm, out_hbm, idxb, rows, semi, semg, semw):
        wid = lax.axis_index("s") * _NC + lax.axis_index("c")
        start = wid * nfull + jnp.minimum(wid, nrem)
        nj = nfull + (wid < nrem).astype(jnp.int32)
        pltpu.sync_copy(idx_hbm.at[start], idxb.at[0])
        pltpu.async_copy(table_hbm.at[idxb.at[0]], rows.at[0], semg)

        def step(j, carry):
            par = j % 2

            @pl.when(j + 1 < nj)
            def _():
                pltpu.async_copy(idx_hbm.at[start + j + 1], idxb.at[1 - par],
                                 semi)

            pltpu.make_async_copy(table_hbm.at[idxb.at[0]], rows.at[0],
                                  semg).wait()
            pltpu.async_copy(rows.at[par], out_hbm.at[start + j], semw)

            @pl.when(j + 1 < nj)
            def _():
                pltpu.make_async_copy(idx_hbm.at[start], idxb.at[0],
                                      semi).wait()

                @pl.when(j >= 1)
                def _():
                    pltpu.make_async_copy(rows.at[0], out_hbm.at[start],
                                          semw).wait()

                pltpu.async_copy(table_hbm.at[idxb.at[1 - par]],
                                 rows.at[1 - par], semg)

            return carry

        lax.fori_loop(0, nj, step, 0)
        pltpu.make_async_copy(rows.at[0], out_hbm.at[start], semw).wait()

        @pl.when(nj >= 2)
        def _():
            pltpu.make_async_copy(rows.at[0], out_hbm.at[start], semw).wait()

    return gk(table, idx2d)


def _sc_scatter_add(e2d, rid_flat, Nm):
    E, D = e2d.shape
    NW = _NC * _NS
    own = (Nm + NW - 1) // NW
    rpt = ((own + 1 + 7) // 8) * 8
    Ep = rid_flat.shape[0]
    OUTER = Ep // 2048
    zeros = jnp.zeros((rpt, D), jnp.float32)

    @functools.partial(
        pl.kernel,
        out_type=jax.ShapeDtypeStruct((NW, rpt, D), jnp.float32),
        mesh=_mesh(),
        scratch_types=[
            pltpu.VMEM((2, 2048), jnp.int32),
            pltpu.VMEM((rpt, D), jnp.float32),
            pltpu.VMEM((1280,), jnp.int32),
            pltpu.VMEM((16,), jnp.int32),
            pltpu.VMEM((2, 16, D), jnp.float32),
            pltpu.SemaphoreType.DMA,
            pltpu.SemaphoreType.DMA,
        ],
        compiler_params=pltpu.CompilerParams(needs_layout_passes=False),
    )
    def sk(zeros_hbm, e_hbm, idx_hbm, out_hbm, idxbuf, acc, plist, midx,
           grow, sem, semx):
        c = lax.axis_index("c")
        s = lax.axis_index("s")
        wid = s * _NC + c
        base = wid * own
        iota16 = lax.iota(jnp.int32, 16)
        pltpu.sync_copy(zeros_hbm, acc)

        def drain_body(n16):
            midx[...] = plist[pl.ds(0, 16)] >> 9
            pltpu.async_copy(e_hbm.at[midx], grow.at[0], sem)

            def batch(b, carry):
                par = b % 2
                pltpu.make_async_copy(e_hbm.at[midx], grow.at[0],
                                      sem).wait()

                @pl.when(b + 1 < n16)
                def _():
                    midx[...] = plist[pl.ds(b * 16 + 16, 16)] >> 9
                    pltpu.async_copy(e_hbm.at[midx], grow.at[1 - par],
                                     sem)

                pk = plist[pl.ds(b * 16, 16)]
                for mm in range(16):
                    li = pk[mm] & 511
                    for k in range(D // 16):
                        plsc.addupdate(acc.at[li, pl.ds(k * 16, 16)],
                                       grow[par, mm, pl.ds(k * 16, 16)])
                return carry

            lax.fori_loop(0, n16, batch, 0)
            plist[pl.ds(0, 16)] = plist[pl.ds(n16 * 16, 16)]

        def drain(m, thresh):
            n16 = m // 16
            do = jnp.logical_and(m >= thresh, n16 > 0)

            @pl.when(do)
            def _():
                drain_body(n16)

            return m - jnp.where(do, n16 * 16, 0)

        pltpu.async_copy(idx_hbm.at[pl.ds(0, 2048)], idxbuf.at[0], semx)

        def outer(o, m):
            op = o % 2
            pltpu.make_async_copy(idx_hbm.at[pl.ds(0, 2048)], idxbuf.at[0],
                                  semx).wait()

            @pl.when(o + 1 < OUTER)
            def _():
                pltpu.async_copy(idx_hbm.at[pl.ds((o + 1) * 2048, 2048)],
                                 idxbuf.at[1 - op], semx)

            def rnd(r, m):
                for vv in range(8):
                    iv = idxbuf[op, pl.ds(r * 128 + vv * 16, 16)]
                    li = iv - base
                    ms = (li >= 0) & (li < own)

                    pos = (o * 2048 + r * 128 + vv * 16) + iota16
                    packed = (pos << 9) | jnp.where(ms, li, 0)
                    cs = plsc.cumsum(ms.astype(jnp.int32))
                    plsc.store_scatter(plist, [m + cs - 1], packed, mask=ms)
                    m = m + plsc.all_reduce_population_count(ms)[0]
                return drain(m, 1024)

            return lax.fori_loop(0, 16, rnd, m)

        m = lax.fori_loop(0, OUTER, outer, jnp.int32(0))
        plist[pl.ds(m, 16)] = jnp.full((16,), own, jnp.int32)
        drain(((m + 15) // 16) * 16, 0)
        pltpu.sync_copy(acc, out_hbm.at[wid])

    out = sk(zeros, e2d, rid_flat)
    return out[:, :own].reshape(NW * own, D)[:Nm]


def _node_proj(v2, Ws, Wr):
    Nm, H = v2.shape
    NB = 1000
    grid = (Nm // NB,)

    def body(v_ref, ws_ref, wr_ref, os_ref, or_ref):
        v = v_ref[...]
        os_ref[...] = jnp.dot(v, ws_ref[...], preferred_element_type=jnp.float32)
        or_ref[...] = jnp.dot(v, wr_ref[...], preferred_element_type=jnp.float32)

    return pl.pallas_call(
        body,
        grid=grid,
        in_specs=[
            pl.BlockSpec((NB, H), lambda i: (i, 0)),
            pl.BlockSpec((H, H), lambda i: (0, 0)),
            pl.BlockSpec((H, H), lambda i: (0, 0)),
        ],
        out_specs=[
            pl.BlockSpec((NB, H), lambda i: (i, 0)),
            pl.BlockSpec((NB, H), lambda i: (i, 0)),
        ],
        out_shape=[
            jax.ShapeDtypeStruct((Nm, H), jnp.float32),
            jax.ShapeDtypeStruct((Nm, H), jnp.float32),
        ],
    )(v2, Ws, Wr)


def _mlp_ln_body(x, extra, w1_ref, w2_ref, b1_ref, g1_ref, be1_ref, b2_ref):
    pre = jnp.dot(x, w1_ref[...], preferred_element_type=jnp.float32)
    pre = pre + extra + b1_ref[...]
    h = pre * jax.nn.sigmoid(pre)
    mu = jnp.mean(h, axis=-1, keepdims=True)
    var = jnp.mean((h - mu) ** 2, axis=-1, keepdims=True)
    h = (h - mu) * lax.rsqrt(var + 1e-5) * g1_ref[...] + be1_ref[...]
    return x + jnp.dot(h, w2_ref[...], preferred_element_type=jnp.float32) + b2_ref[...]


def _edge_mlp(e2, gs, gr, W1e, W2, b1, g1, be1, b2):
    E, H = e2.shape
    EB = 640
    grid = (E // EB,)

    def body(e_ref, gs_ref, gr_ref, w1_ref, w2_ref, b1_ref, g1_ref, be1_ref,
             b2_ref, o_ref):
        o_ref[...] = _mlp_ln_body(e_ref[...], gs_ref[...] + gr_ref[...],
                                  w1_ref, w2_ref, b1_ref, g1_ref, be1_ref, b2_ref)

    row = pl.BlockSpec((EB, H), lambda i: (i, 0))
    mat = pl.BlockSpec((H, H), lambda i: (0, 0))
    vec = pl.BlockSpec((1, H), lambda i: (0, 0))
    return pl.pallas_call(
        body,
        grid=grid,
        in_specs=[row, row, row, mat, mat, vec, vec, vec, vec],
        out_specs=row,
        out_shape=jax.ShapeDtypeStruct((E, H), jnp.float32),
    )(e2, gs, gr, W1e, W2, b1, g1, be1, b2)


def _node_mlp(v2, agg, W1v, W1a, W2, b1, g1, be1, b2):
    Nm, H = v2.shape
    NB = 1000
    grid = (Nm // NB,)

    def body(v_ref, a_ref, w1_ref, w1a_ref, w2_ref, b1_ref, g1_ref,
             be1_ref, b2_ref, o_ref):
        extra = jnp.dot(a_ref[...], w1a_ref[...],
                        preferred_element_type=jnp.float32)
        o_ref[...] = _mlp_ln_body(v_ref[...], extra, w1_ref, w2_ref, b1_ref,
                                  g1_ref, be1_ref, b2_ref)

    row = pl.BlockSpec((NB, H), lambda i: (i, 0))
    mat = pl.BlockSpec((H, H), lambda i: (0, 0))
    vec = pl.BlockSpec((1, H), lambda i: (0, 0))
    return pl.pallas_call(
        body,
        grid=grid,
        in_specs=[row, row, mat, mat, mat, vec, vec, vec, vec],
        out_specs=row,
        out_shape=jax.ShapeDtypeStruct((Nm, H), jnp.float32),
    )(v2, agg, W1v, W1a, W2, b1, g1, be1, b2)


def kernel(vM, eM, senders, receivers, W1m, b1m, g1m, be1m, W2m, b2m,
           W1n, b1n, g1n, be1n, W2n, b2n):
    B, Nm, H = vM.shape
    E = eM.shape[1]
    v2 = vM[0]
    e2 = eM[0]
    sid = senders.astype(jnp.int32).reshape(E // _C, _C)
    rid = receivers.astype(jnp.int32).reshape(E // _C, _C)

    b1m_ = b1m.reshape(1, H)
    g1m_ = g1m.reshape(1, H)
    be1m_ = be1m.reshape(1, H)
    b2m_ = b2m.reshape(1, H)
    b1n_ = b1n.reshape(1, H)
    g1n_ = g1n.reshape(1, H)
    be1n_ = be1n.reshape(1, H)
    b2n_ = b2n.reshape(1, H)

    Ps, Pr = _node_proj(v2, W1m[H:2 * H], W1m[2 * H:])
    gs = _sc_gather(Ps, sid).reshape(E, H)
    gr = _sc_gather(Pr, rid).reshape(E, H)
    e2out = _edge_mlp(e2, gs, gr, W1m[:H], W2m, b1m_, g1m_, be1m_, b2m_)
    r32 = receivers.astype(jnp.int32)
    Ep = ((E + 2047) // 2048) * 2048
    rid_flat = jnp.concatenate(
        [r32, jnp.full((Ep - E,), 2 ** 20, jnp.int32)])
    agg = _sc_scatter_add(e2out, rid_flat, Nm)
    v2out = _node_mlp(v2, agg, W1n[:H], W1n[H:], W2n,
                      b1n_, g1n_, be1n_, b2n_)
    return (v2out.reshape(B, Nm, H), e2out.reshape(B, E, H))


# speedup vs baseline: 1.4029x; 1.0092x over previous
import functools

import jax
import jax.numpy as jnp
from jax import lax
from jax.experimental import pallas as pl
from jax.experimental.pallas import tpu as pltpu
from jax.experimental.pallas import tpu_sc as plsc

_NC = 2
_NS = 16
_C = 128


def _mesh():
    return plsc.VectorSubcoreMesh(core_axis_name="c", subcore_axis_name="s")


def _sc_gather2(tabS, tabR, sid2d, rid2d):
    R, C = sid2d.shape
    D = tabS.shape[1]
    NW = _NC * _NS
    nfull, nrem = R // NW, R % NW

    @functools.partial(
        pl.kernel,
        out_type=[jax.ShapeDtypeStruct((R, C, D), jnp.float32),
                  jax.ShapeDtypeStruct((R, C, D), jnp.float32)],
        mesh=_mesh(),
        scratch_types=[
            pltpu.VMEM((2, C), jnp.int32),
            pltpu.VMEM((2, C, D), jnp.float32),
            pltpu.SemaphoreType.DMA,
            pltpu.SemaphoreType.DMA,
            pltpu.SemaphoreType.DMA,
        ],
    )
    def gk(tabS_hbm, tabR_hbm, sid_hbm, rid_hbm, outS_hbm, outR_hbm,
           idxb, rows, semi, semg, semw):
        wid = lax.axis_index("s") * _NC + lax.axis_index("c")
        start = wid * nfull + jnp.minimum(wid, nrem)
        nj = nfull + (wid < nrem).astype(jnp.int32)
        nt = 2 * nj
        pltpu.sync_copy(sid_hbm.at[start], idxb.at[0])
        pltpu.async_copy(tabS_hbm.at[idxb.at[0]], rows.at[0], semg)

        def step(t, carry):
            q = t % 2
            j = start + t // 2

            @pl.when(jnp.logical_and(q == 0, t + 1 < nt))
            def _():
                pltpu.async_copy(rid_hbm.at[j], idxb.at[1], semi)

            @pl.when(jnp.logical_and(q == 1, t + 1 < nt))
            def _():
                pltpu.async_copy(sid_hbm.at[j + 1], idxb.at[0], semi)

            pltpu.make_async_copy(tabS_hbm.at[idxb.at[0]], rows.at[0],
                                  semg).wait()

            @pl.when(q == 0)
            def _():
                pltpu.async_copy(rows.at[0], outS_hbm.at[j], semw)

            @pl.when(q == 1)
            def _():
                pltpu.async_copy(rows.at[1], outR_hbm.at[j], semw)

            @pl.when(t + 1 < nt)
            def _():
                pltpu.make_async_copy(sid_hbm.at[start], idxb.at[0],
                                      semi).wait()

                @pl.when(t >= 1)
                def _():
                    pltpu.make_async_copy(rows.at[0], outS_hbm.at[start],
                                          semw).wait()

                @pl.when(q == 0)
                def _():
                    pltpu.async_copy(tabR_hbm.at[idxb.at[1]], rows.at[1],
                                     semg)

                @pl.when(q == 1)
                def _():
                    pltpu.async_copy(tabS_hbm.at[idxb.at[0]], rows.at[0],
                                     semg)

            return carry

        lax.fori_loop(0, nt, step, 0)
        pltpu.make_async_copy(rows.at[0], outS_hbm.at[start], semw).wait()
        pltpu.make_async_copy(rows.at[0], outS_hbm.at[start], semw).wait()

    return gk(tabS, tabR, sid2d, rid2d)


def _sc_scatter_add(e2d, rid_flat, Nm):
    E, D = e2d.shape
    NW = _NC * _NS
    own = (Nm + NW - 1) // NW
    rpt = ((own + 1 + 7) // 8) * 8
    Ep = rid_flat.shape[0]
    OUTER = Ep // 2048
    zeros = jnp.zeros((rpt, D), jnp.float32)

    @functools.partial(
        pl.kernel,
        out_type=jax.ShapeDtypeStruct((NW, rpt, D), jnp.float32),
        mesh=_mesh(),
        scratch_types=[
            pltpu.VMEM((2, 2048), jnp.int32),
            pltpu.VMEM((rpt, D), jnp.float32),
            pltpu.VMEM((1280,), jnp.int32),
            pltpu.VMEM((16,), jnp.int32),
            pltpu.VMEM((2, 16, D), jnp.float32),
            pltpu.SemaphoreType.DMA,
            pltpu.SemaphoreType.DMA,
        ],
        compiler_params=pltpu.CompilerParams(needs_layout_passes=False),
    )
    def sk(zeros_hbm, e_hbm, idx_hbm, out_hbm, idxbuf, acc, plist, midx,
           grow, sem, semx):
        c = lax.axis_index("c")
        s = lax.axis_index("s")
        wid = s * _NC + c
        base = wid * own
        iota16 = lax.iota(jnp.int32, 16)
        pltpu.sync_copy(zeros_hbm, acc)

        def drain_body(n16):
            midx[...] = plist[pl.ds(0, 16)] >> 9
            pltpu.async_copy(e_hbm.at[midx], grow.at[0], sem)

            def batch(b, carry):
                par = b % 2
                pltpu.make_async_copy(e_hbm.at[midx], grow.at[0],
                                      sem).wait()

                @pl.when(b + 1 < n16)
                def _():
                    midx[...] = plist[pl.ds(b * 16 + 16, 16)] >> 9
                    pltpu.async_copy(e_hbm.at[midx], grow.at[1 - par],
                                     sem)

                pk = plist[pl.ds(b * 16, 16)]
                for mm in range(16):
                    li = pk[mm] & 511
                    for k in range(D // 16):
                        plsc.addupdate(acc.at[li, pl.ds(k * 16, 16)],
                                       grow[par, mm, pl.ds(k * 16, 16)])
                return carry

            lax.fori_loop(0, n16, batch, 0)
            plist[pl.ds(0, 16)] = plist[pl.ds(n16 * 16, 16)]

        def drain(m, thresh):
            n16 = m // 16
            do = jnp.logical_and(m >= thresh, n16 > 0)

            @pl.when(do)
            def _():
                drain_body(n16)

            return m - jnp.where(do, n16 * 16, 0)

        pltpu.async_copy(idx_hbm.at[pl.ds(0, 2048)], idxbuf.at[0], semx)

        def outer(o, m):
            op = o % 2
            pltpu.make_async_copy(idx_hbm.at[pl.ds(0, 2048)], idxbuf.at[0],
                                  semx).wait()

            @pl.when(o + 1 < OUTER)
            def _():
                pltpu.async_copy(idx_hbm.at[pl.ds((o + 1) * 2048, 2048)],
                                 idxbuf.at[1 - op], semx)

            def rnd(r, m):
                for vv in range(8):
                    iv = idxbuf[op, pl.ds(r * 128 + vv * 16, 16)]
                    li = iv - base
                    ms = (li >= 0) & (li < own)

                    pos = (o * 2048 + r * 128 + vv * 16) + iota16
                    packed = (pos << 9) | jnp.where(ms, li, 0)
                    cs = plsc.cumsum(ms.astype(jnp.int32))
                    plsc.store_scatter(plist, [m + cs - 1], packed, mask=ms)
                    m = m + plsc.all_reduce_population_count(ms)[0]
                return drain(m, 1024)

            return lax.fori_loop(0, 16, rnd, m)

        m = lax.fori_loop(0, OUTER, outer, jnp.int32(0))
        plist[pl.ds(m, 16)] = jnp.full((16,), own, jnp.int32)
        drain(((m + 15) // 16) * 16, 0)
        pltpu.sync_copy(acc, out_hbm.at[wid])

    out = sk(zeros, e2d, rid_flat)
    return out[:, :own].reshape(NW * own, D)[:Nm]


def _node_proj(v2, Ws, Wr):
    Nm, H = v2.shape
    NB = 1000
    grid = (Nm // NB,)

    def body(v_ref, ws_ref, wr_ref, os_ref, or_ref):
        v = v_ref[...]
        os_ref[...] = jnp.dot(v, ws_ref[...], preferred_element_type=jnp.float32)
        or_ref[...] = jnp.dot(v, wr_ref[...], preferred_element_type=jnp.float32)

    return pl.pallas_call(
        body,
        grid=grid,
        in_specs=[
            pl.BlockSpec((NB, H), lambda i: (i, 0)),
            pl.BlockSpec((H, H), lambda i: (0, 0)),
            pl.BlockSpec((H, H), lambda i: (0, 0)),
        ],
        out_specs=[
            pl.BlockSpec((NB, H), lambda i: (i, 0)),
            pl.BlockSpec((NB, H), lambda i: (i, 0)),
        ],
        out_shape=[
            jax.ShapeDtypeStruct((Nm, H), jnp.float32),
            jax.ShapeDtypeStruct((Nm, H), jnp.float32),
        ],
    )(v2, Ws, Wr)


def _mlp_ln_body(x, extra, w1_ref, w2_ref, b1_ref, g1_ref, be1_ref, b2_ref):
    pre = jnp.dot(x, w1_ref[...], preferred_element_type=jnp.float32)
    pre = pre + extra + b1_ref[...]
    h = pre * jax.nn.sigmoid(pre)
    mu = jnp.mean(h, axis=-1, keepdims=True)
    var = jnp.mean((h - mu) ** 2, axis=-1, keepdims=True)
    h = (h - mu) * lax.rsqrt(var + 1e-5) * g1_ref[...] + be1_ref[...]
    return x + jnp.dot(h, w2_ref[...], preferred_element_type=jnp.float32) + b2_ref[...]


def _edge_mlp(e2, gs, gr, W1e, W2, b1, g1, be1, b2):
    E, H = e2.shape
    EB = 640
    grid = (E // EB,)

    def body(e_ref, gs_ref, gr_ref, w1_ref, w2_ref, b1_ref, g1_ref, be1_ref,
             b2_ref, o_ref):
        o_ref[...] = _mlp_ln_body(e_ref[...], gs_ref[...] + gr_ref[...],
                                  w1_ref, w2_ref, b1_ref, g1_ref, be1_ref, b2_ref)

    row = pl.BlockSpec((EB, H), lambda i: (i, 0))
    mat = pl.BlockSpec((H, H), lambda i: (0, 0))
    vec = pl.BlockSpec((1, H), lambda i: (0, 0))
    return pl.pallas_call(
        body,
        grid=grid,
        in_specs=[row, row, row, mat, mat, vec, vec, vec, vec],
        out_specs=row,
        out_shape=jax.ShapeDtypeStruct((E, H), jnp.float32),
    )(e2, gs, gr, W1e, W2, b1, g1, be1, b2)


def _node_mlp(v2, agg, W1v, W1a, W2, b1, g1, be1, b2):
    Nm, H = v2.shape
    NB = 1000
    grid = (Nm // NB,)

    def body(v_ref, a_ref, w1_ref, w1a_ref, w2_ref, b1_ref, g1_ref,
             be1_ref, b2_ref, o_ref):
        extra = jnp.dot(a_ref[...], w1a_ref[...],
                        preferred_element_type=jnp.float32)
        o_ref[...] = _mlp_ln_body(v_ref[...], extra, w1_ref, w2_ref, b1_ref,
                                  g1_ref, be1_ref, b2_ref)

    row = pl.BlockSpec((NB, H), lambda i: (i, 0))
    mat = pl.BlockSpec((H, H), lambda i: (0, 0))
    vec = pl.BlockSpec((1, H), lambda i: (0, 0))
    return pl.pallas_call(
        body,
        grid=grid,
        in_specs=[row, row, mat, mat, mat, vec, vec, vec, vec],
        out_specs=row,
        out_shape=jax.ShapeDtypeStruct((Nm, H), jnp.float32),
    )(v2, agg, W1v, W1a, W2, b1, g1, be1, b2)


def kernel(vM, eM, senders, receivers, W1m, b1m, g1m, be1m, W2m, b2m,
           W1n, b1n, g1n, be1n, W2n, b2n):
    B, Nm, H = vM.shape
    E = eM.shape[1]
    v2 = vM[0]
    e2 = eM[0]
    sid = senders.astype(jnp.int32).reshape(E // _C, _C)
    rid = receivers.astype(jnp.int32).reshape(E // _C, _C)

    b1m_ = b1m.reshape(1, H)
    g1m_ = g1m.reshape(1, H)
    be1m_ = be1m.reshape(1, H)
    b2m_ = b2m.reshape(1, H)
    b1n_ = b1n.reshape(1, H)
    g1n_ = g1n.reshape(1, H)
    be1n_ = be1n.reshape(1, H)
    b2n_ = b2n.reshape(1, H)

    Ps, Pr = _node_proj(v2, W1m[H:2 * H], W1m[2 * H:])
    gs3, gr3 = _sc_gather2(Ps, Pr, sid, rid)
    gs = gs3.reshape(E, H)
    gr = gr3.reshape(E, H)
    e2out = _edge_mlp(e2, gs, gr, W1m[:H], W2m, b1m_, g1m_, be1m_, b2m_)
    r32 = receivers.astype(jnp.int32)
    Ep = ((E + 2047) // 2048) * 2048
    rid_flat = jnp.concatenate(
        [r32, jnp.full((Ep - E,), 2 ** 20, jnp.int32)])
    agg = _sc_scatter_add(e2out, rid_flat, Nm)
    v2out = _node_mlp(v2, agg, W1n[:H], W1n[H:], W2n,
                      b1n_, g1n_, be1n_, b2n_)
    return (v2out.reshape(B, Nm, H), e2out.reshape(B, E, H))


# speedup vs baseline: 1.5312x; 1.0915x over previous
import functools

import jax
import jax.numpy as jnp
from jax import lax
from jax.experimental import pallas as pl
from jax.experimental.pallas import tpu as pltpu
from jax.experimental.pallas import tpu_sc as plsc

_NC = 2
_NS = 16
_C = 128


def _mesh():
    return plsc.VectorSubcoreMesh(core_axis_name="c", subcore_axis_name="s")


def _sc_gather2(tabS, tabR, sid2d, rid2d):
    R, C = sid2d.shape
    D = tabS.shape[1]
    NW = _NC * _NS
    nfull, nrem = R // NW, R % NW

    @functools.partial(
        pl.kernel,
        out_type=[jax.ShapeDtypeStruct((R, C, D), jnp.float32),
                  jax.ShapeDtypeStruct((R, C, D), jnp.float32)],
        mesh=_mesh(),
        scratch_types=[
            pltpu.VMEM((2, C), jnp.int32),
            pltpu.VMEM((2, C, D), jnp.float32),
            pltpu.SemaphoreType.DMA,
            pltpu.SemaphoreType.DMA,
            pltpu.SemaphoreType.DMA,
        ],
    )
    def gk(tabS_hbm, tabR_hbm, sid_hbm, rid_hbm, outS_hbm, outR_hbm,
           idxb, rows, semi, semg, semw):
        wid = lax.axis_index("s") * _NC + lax.axis_index("c")
        start = wid * nfull + jnp.minimum(wid, nrem)
        nj = nfull + (wid < nrem).astype(jnp.int32)
        nt = 2 * nj
        pltpu.sync_copy(sid_hbm.at[start], idxb.at[0])
        pltpu.async_copy(tabS_hbm.at[idxb.at[0]], rows.at[0], semg)

        def step(t, carry):
            q = t % 2
            j = start + t // 2

            @pl.when(jnp.logical_and(q == 0, t + 1 < nt))
            def _():
                pltpu.async_copy(rid_hbm.at[j], idxb.at[1], semi)

            @pl.when(jnp.logical_and(q == 1, t + 1 < nt))
            def _():
                pltpu.async_copy(sid_hbm.at[j + 1], idxb.at[0], semi)

            pltpu.make_async_copy(tabS_hbm.at[idxb.at[0]], rows.at[0],
                                  semg).wait()

            @pl.when(q == 0)
            def _():
                pltpu.async_copy(rows.at[0], outS_hbm.at[j], semw)

            @pl.when(q == 1)
            def _():
                pltpu.async_copy(rows.at[1], outR_hbm.at[j], semw)

            @pl.when(t + 1 < nt)
            def _():
                pltpu.make_async_copy(sid_hbm.at[start], idxb.at[0],
                                      semi).wait()

                @pl.when(t >= 1)
                def _():
                    pltpu.make_async_copy(rows.at[0], outS_hbm.at[start],
                                          semw).wait()

                @pl.when(q == 0)
                def _():
                    pltpu.async_copy(tabR_hbm.at[idxb.at[1]], rows.at[1],
                                     semg)

                @pl.when(q == 1)
                def _():
                    pltpu.async_copy(tabS_hbm.at[idxb.at[0]], rows.at[0],
                                     semg)

            return carry

        lax.fori_loop(0, nt, step, 0)
        pltpu.make_async_copy(rows.at[0], outS_hbm.at[start], semw).wait()
        pltpu.make_async_copy(rows.at[0], outS_hbm.at[start], semw).wait()

    return gk(tabS, tabR, sid2d, rid2d)


def _sc_scatter_add(e2d, rid_flat, Nm):
    E, D = e2d.shape
    NW = _NC * _NS
    own = (Nm + NW - 1) // NW
    rpt = ((own + 1 + 7) // 8) * 8
    Ep = rid_flat.shape[0]
    OUTER = Ep // 2048
    zeros = jnp.zeros((rpt, D), jnp.float32)

    @functools.partial(
        pl.kernel,
        out_type=jax.ShapeDtypeStruct((NW, rpt, D), jnp.float32),
        mesh=_mesh(),
        scratch_types=[
            pltpu.VMEM((2, 2048), jnp.int32),
            pltpu.VMEM((rpt, D), jnp.float32),
            pltpu.VMEM((1280,), jnp.int32),
            pltpu.VMEM((16,), jnp.int32),
            pltpu.VMEM((2, 16, D), jnp.float32),
            pltpu.SemaphoreType.DMA,
            pltpu.SemaphoreType.DMA,
        ],
        compiler_params=pltpu.CompilerParams(needs_layout_passes=False),
    )
    def sk(zeros_hbm, e_hbm, idx_hbm, out_hbm, idxbuf, acc, plist, midx,
           grow, sem, semx):
        c = lax.axis_index("c")
        s = lax.axis_index("s")
        wid = s * _NC + c
        base = wid * own
        iota16 = lax.iota(jnp.int32, 16)
        pltpu.sync_copy(zeros_hbm, acc)

        def drain_body(n16):
            midx[...] = plist[pl.ds(0, 16)] >> 9
            pltpu.async_copy(e_hbm.at[midx], grow.at[0], sem)

            def batch(b, carry):
                par = b % 2
                pltpu.make_async_copy(e_hbm.at[midx], grow.at[0],
                                      sem).wait()

                @pl.when(b + 1 < n16)
                def _():
                    midx[...] = plist[pl.ds(b * 16 + 16, 16)] >> 9
                    pltpu.async_copy(e_hbm.at[midx], grow.at[1 - par],
                                     sem)

                pk = plist[pl.ds(b * 16, 16)]
                for mm in range(16):
                    li = pk[mm] & 511
                    for k in range(D // 16):
                        plsc.addupdate(acc.at[li, pl.ds(k * 16, 16)],
                                       grow[par, mm, pl.ds(k * 16, 16)])
                return carry

            lax.fori_loop(0, n16, batch, 0)
            plist[pl.ds(0, 16)] = plist[pl.ds(n16 * 16, 16)]

        def drain(m, thresh):
            n16 = m // 16
            do = jnp.logical_and(m >= thresh, n16 > 0)

            @pl.when(do)
            def _():
                drain_body(n16)

            return m - jnp.where(do, n16 * 16, 0)

        pltpu.async_copy(idx_hbm.at[pl.ds(0, 2048)], idxbuf.at[0], semx)

        def outer(o, m):
            op = o % 2
            pltpu.make_async_copy(idx_hbm.at[pl.ds(0, 2048)], idxbuf.at[0],
                                  semx).wait()

            @pl.when(o + 1 < OUTER)
            def _():
                pltpu.async_copy(idx_hbm.at[pl.ds((o + 1) * 2048, 2048)],
                                 idxbuf.at[1 - op], semx)

            def rnd(r, m):
                for vv in range(8):
                    iv = idxbuf[op, pl.ds(r * 128 + vv * 16, 16)]
                    li = iv - base
                    ms = (li >= 0) & (li < own)

                    pos = (o * 2048 + r * 128 + vv * 16) + iota16
                    packed = (pos << 9) | jnp.where(ms, li, 0)
                    cs = plsc.cumsum(ms.astype(jnp.int32))
                    plsc.store_scatter(plist, [m + cs - 1], packed, mask=ms)
                    m = m + plsc.all_reduce_population_count(ms)[0]
                return drain(m, 1024)

            return lax.fori_loop(0, 16, rnd, m)

        m = lax.fori_loop(0, OUTER, outer, jnp.int32(0))
        plist[pl.ds(m, 16)] = jnp.full((16,), own, jnp.int32)
        drain(((m + 15) // 16) * 16, 0)
        pltpu.sync_copy(acc, out_hbm.at[wid])

    out = sk(zeros, e2d, rid_flat)
    return out[:, :own].reshape(NW * own, D)[:Nm]


def _node_proj(v2, Ws, Wr):
    Nm, H = v2.shape
    NB = 1000
    grid = (Nm // NB,)

    def body(v_ref, ws_ref, wr_ref, os_ref, or_ref):
        v = v_ref[...]
        os_ref[...] = jnp.dot(v, ws_ref[...], preferred_element_type=jnp.float32)
        or_ref[...] = jnp.dot(v, wr_ref[...], preferred_element_type=jnp.float32)

    return pl.pallas_call(
        body,
        grid=grid,
        in_specs=[
            pl.BlockSpec((NB, H), lambda i: (i, 0)),
            pl.BlockSpec((H, H), lambda i: (0, 0)),
            pl.BlockSpec((H, H), lambda i: (0, 0)),
        ],
        out_specs=[
            pl.BlockSpec((NB, H), lambda i: (i, 0)),
            pl.BlockSpec((NB, H), lambda i: (i, 0)),
        ],
        out_shape=[
            jax.ShapeDtypeStruct((Nm, H), jnp.float32),
            jax.ShapeDtypeStruct((Nm, H), jnp.float32),
        ],
    )(v2, Ws, Wr)


def _mlp_ln_body(x, extra, w1_ref, w2_ref, b1_ref, g1_ref, be1_ref, b2_ref):
    pre = jnp.dot(x, w1_ref[...], preferred_element_type=jnp.float32)
    pre = pre + extra + b1_ref[...]
    h = pre * jax.nn.sigmoid(pre)
    mu = jnp.mean(h, axis=-1, keepdims=True)
    var = jnp.mean((h - mu) ** 2, axis=-1, keepdims=True)
    h = (h - mu) * lax.rsqrt(var + 1e-5) * g1_ref[...] + be1_ref[...]
    return x + jnp.dot(h, w2_ref[...], preferred_element_type=jnp.float32) + b2_ref[...]


def _edge_mlp(e2, gs, gr, W1e, W2, b1, g1, be1, b2):
    E, H = e2.shape
    EB = 1280
    grid = (E // EB,)

    def body(e_ref, gs_ref, gr_ref, w1_ref, w2_ref, b1_ref, g1_ref, be1_ref,
             b2_ref, o_ref):
        o_ref[...] = _mlp_ln_body(e_ref[...], gs_ref[...] + gr_ref[...],
                                  w1_ref, w2_ref, b1_ref, g1_ref, be1_ref, b2_ref)

    row = pl.BlockSpec((EB, H), lambda i: (i, 0))
    mat = pl.BlockSpec((H, H), lambda i: (0, 0))
    vec = pl.BlockSpec((1, H), lambda i: (0, 0))
    return pl.pallas_call(
        body,
        grid=grid,
        in_specs=[row, row, row, mat, mat, vec, vec, vec, vec],
        out_specs=row,
        out_shape=jax.ShapeDtypeStruct((E, H), jnp.float32),
    )(e2, gs, gr, W1e, W2, b1, g1, be1, b2)


def _node_mlp(v2, agg, W1v, W1a, W2, b1, g1, be1, b2):
    Nm, H = v2.shape
    NB = 1000
    grid = (Nm // NB,)

    def body(v_ref, a_ref, w1_ref, w1a_ref, w2_ref, b1_ref, g1_ref,
             be1_ref, b2_ref, o_ref):
        extra = jnp.dot(a_ref[...], w1a_ref[...],
                        preferred_element_type=jnp.float32)
        o_ref[...] = _mlp_ln_body(v_ref[...], extra, w1_ref, w2_ref, b1_ref,
                                  g1_ref, be1_ref, b2_ref)

    row = pl.BlockSpec((NB, H), lambda i: (i, 0))
    mat = pl.BlockSpec((H, H), lambda i: (0, 0))
    vec = pl.BlockSpec((1, H), lambda i: (0, 0))
    return pl.pallas_call(
        body,
        grid=grid,
        in_specs=[row, row, mat, mat, mat, vec, vec, vec, vec],
        out_specs=row,
        out_shape=jax.ShapeDtypeStruct((Nm, H), jnp.float32),
    )(v2, agg, W1v, W1a, W2, b1, g1, be1, b2)


def kernel(vM, eM, senders, receivers, W1m, b1m, g1m, be1m, W2m, b2m,
           W1n, b1n, g1n, be1n, W2n, b2n):
    B, Nm, H = vM.shape
    E = eM.shape[1]
    v2 = vM[0]
    e2 = eM[0]
    sid = senders.astype(jnp.int32).reshape(E // _C, _C)
    rid = receivers.astype(jnp.int32).reshape(E // _C, _C)

    b1m_ = b1m.reshape(1, H)
    g1m_ = g1m.reshape(1, H)
    be1m_ = be1m.reshape(1, H)
    b2m_ = b2m.reshape(1, H)
    b1n_ = b1n.reshape(1, H)
    g1n_ = g1n.reshape(1, H)
    be1n_ = be1n.reshape(1, H)
    b2n_ = b2n.reshape(1, H)

    Ps, Pr = _node_proj(v2, W1m[H:2 * H], W1m[2 * H:])
    gs3, gr3 = _sc_gather2(Ps, Pr, sid, rid)
    gs = gs3.reshape(E, H)
    gr = gr3.reshape(E, H)
    e2out = _edge_mlp(e2, gs, gr, W1m[:H], W2m, b1m_, g1m_, be1m_, b2m_)
    r32 = receivers.astype(jnp.int32)
    Ep = ((E + 2047) // 2048) * 2048
    rid_flat = jnp.concatenate(
        [r32, jnp.full((Ep - E,), 2 ** 20, jnp.int32)])
    agg = _sc_scatter_add(e2out, rid_flat, Nm)
    v2out = _node_mlp(v2, agg, W1n[:H], W1n[H:], W2n,
                      b1n_, g1n_, be1n_, b2n_)
    return (v2out.reshape(B, Nm, H), e2out.reshape(B, E, H))
